# baseline (device time: 23788 ns/iter reference)
import jax
import jax.numpy as jnp
from jax import lax
from jax.experimental import pallas as pl
from jax.experimental.pallas import tpu as pltpu

B, Sq, Skv, Hq, Dh = 2, 128, 128, 16, 64
H_LOC = 4
D_MODEL = 512


def kernel(x, Wq, K_ext, V_ext, Wo):
    my_pos = lax.axis_index("i")
    k_loc = lax.dynamic_slice_in_dim(K_ext, my_pos * H_LOC, H_LOC, axis=2)
    v_loc = lax.dynamic_slice_in_dim(V_ext, my_pos * H_LOC, H_LOC, axis=2)

    def body(x_ref, wq_ref, k_ref, v_ref, wo_ref, out_ref,
             recv_ref, send_sems, recv_sems):
        my_i = lax.axis_index("i")
        p1 = my_i ^ 1
        p2 = 3 - my_i

        barrier = pltpu.get_barrier_semaphore()
        for p in (p1, p2):
            pl.semaphore_signal(
                barrier, inc=1,
                device_id=(p,), device_id_type=pl.DeviceIdType.MESH,
            )
        pl.semaphore_wait(barrier, 2)

        q_blk = lax.broadcasted_iota(jnp.int32, (Sq, Skv), 0) // 64
        k_blk = lax.broadcasted_iota(jnp.int32, (Sq, Skv), 1) // 64
        bias = jnp.where(k_blk <= q_blk, 0.0, -1e9).astype(jnp.float32)

        wq = wq_ref[...].astype(jnp.bfloat16)
        wo = wo_ref[...].astype(jnp.bfloat16)

        for b in range(B):
            xb = x_ref[b].astype(jnp.bfloat16)
            q = lax.dot(xb, wq, preferred_element_type=jnp.float32)
            q = q.astype(jnp.bfloat16)
            kb = k_ref[b].astype(jnp.bfloat16)
            vb = v_ref[b].astype(jnp.bfloat16)
            ctx_heads = []
            for h in range(H_LOC):
                qh = q[:, h * Dh:(h + 1) * Dh]
                kh = kb[:, h, :]
                vh = vb[:, h, :]
                s = lax.dot_general(
                    qh, kh, (((1,), (1,)), ((), ())),
                    preferred_element_type=jnp.float32,
                ) * 0.125 + bias
                m = jnp.max(s, axis=-1, keepdims=True)
                w = jnp.exp(s - m)
                w = w / jnp.sum(w, axis=-1, keepdims=True)
                ctx_heads.append(
                    lax.dot(w.astype(jnp.bfloat16), vh,
                            preferred_element_type=jnp.float32)
                )
            ctx = jnp.concatenate(ctx_heads, axis=1)
            out_ref[b] = lax.dot(
                ctx.astype(jnp.bfloat16), wo,
                preferred_element_type=jnp.float32,
            )

        for s_idx, p in enumerate((p1, p2)):
            rdma = pltpu.make_async_remote_copy(
                src_ref=out_ref,
                dst_ref=recv_ref.at[s_idx],
                send_sem=send_sems.at[s_idx],
                recv_sem=recv_sems.at[s_idx],
                device_id=(p,),
                device_id_type=pl.DeviceIdType.MESH,
            )
            rdma.start()
            rdma.wait()
            out_ref[...] = out_ref[...] + recv_ref[s_idx]

    return pl.pallas_call(
        body,
        out_shape=jax.ShapeDtypeStruct((B, Sq, D_MODEL), jnp.float32),
        in_specs=[pl.BlockSpec(memory_space=pltpu.VMEM)] * 5,
        out_specs=pl.BlockSpec(memory_space=pltpu.VMEM),
        scratch_shapes=[
            pltpu.VMEM((2, B, Sq, D_MODEL), jnp.float32),
            pltpu.SemaphoreType.DMA((2,)),
            pltpu.SemaphoreType.DMA((2,)),
        ],
        compiler_params=pltpu.CompilerParams(collective_id=0),
    )(x, Wq, k_loc, v_loc, Wo)


# device time: 18201 ns/iter; 1.3070x vs baseline; 1.3070x over previous
import jax
import jax.numpy as jnp
from jax import lax
from jax.experimental import pallas as pl
from jax.experimental.pallas import tpu as pltpu

B, Sq, Skv, Hq, Dh = 2, 128, 128, 16, 64
H_LOC = 4
D_MODEL = 512


def kernel(x, Wq, K_ext, V_ext, Wo):
    my_pos = lax.axis_index("i")
    k_loc = lax.dynamic_slice_in_dim(K_ext, my_pos * H_LOC, H_LOC, axis=2)
    v_loc = lax.dynamic_slice_in_dim(V_ext, my_pos * H_LOC, H_LOC, axis=2)

    def body(x_ref, wq_ref, k_ref, v_ref, wo_ref, out_ref,
             send_ref, recv_ref, send_sems, recv_sems):
        my_i = lax.axis_index("i")
        p1 = my_i ^ 1
        p2 = 3 - my_i

        barrier = pltpu.get_barrier_semaphore()
        for p in (p1, p2):
            pl.semaphore_signal(
                barrier, inc=1,
                device_id=(p,), device_id_type=pl.DeviceIdType.MESH,
            )
        pl.semaphore_wait(barrier, 2)

        q_blk = lax.broadcasted_iota(jnp.int32, (Sq, Skv), 0) // 64
        k_blk = lax.broadcasted_iota(jnp.int32, (Sq, Skv), 1) // 64
        bias = jnp.where(k_blk <= q_blk, 0.0, -1e9).astype(jnp.float32)

        wq = wq_ref[...].astype(jnp.bfloat16)
        wo = wo_ref[...].astype(jnp.bfloat16)

        for b in range(B):
            xb = x_ref[b].astype(jnp.bfloat16)
            q = lax.dot(xb, wq, preferred_element_type=jnp.float32)
            q = q.astype(jnp.bfloat16)
            kb = k_ref[b].astype(jnp.bfloat16)
            vb = v_ref[b].astype(jnp.bfloat16)
            ctx_heads = []
            for h in range(H_LOC):
                qh = q[:, h * Dh:(h + 1) * Dh]
                kh = kb[:, h, :]
                vh = vb[:, h, :]
                s = lax.dot_general(
                    qh, kh, (((1,), (1,)), ((), ())),
                    preferred_element_type=jnp.float32,
                ) * 0.125 + bias
                m = jnp.max(s, axis=-1, keepdims=True)
                w = jnp.exp(s - m)
                w = w / jnp.sum(w, axis=-1, keepdims=True)
                ctx_heads.append(
                    lax.dot(w.astype(jnp.bfloat16), vh,
                            preferred_element_type=jnp.float32)
                )
            ctx = jnp.concatenate(ctx_heads, axis=1)
            out_ref[b] = lax.dot(
                ctx.astype(jnp.bfloat16), wo,
                preferred_element_type=jnp.float32,
            )

        for s_idx, p in enumerate((p1, p2)):
            send_ref[s_idx] = out_ref[...].astype(jnp.bfloat16)
            rdma = pltpu.make_async_remote_copy(
                src_ref=send_ref.at[s_idx],
                dst_ref=recv_ref.at[s_idx],
                send_sem=send_sems.at[s_idx],
                recv_sem=recv_sems.at[s_idx],
                device_id=(p,),
                device_id_type=pl.DeviceIdType.MESH,
            )
            rdma.start()
            rdma.wait()
            out_ref[...] = out_ref[...] + recv_ref[s_idx].astype(jnp.float32)

    return pl.pallas_call(
        body,
        out_shape=jax.ShapeDtypeStruct((B, Sq, D_MODEL), jnp.float32),
        in_specs=[pl.BlockSpec(memory_space=pltpu.VMEM)] * 5,
        out_specs=pl.BlockSpec(memory_space=pltpu.VMEM),
        scratch_shapes=[
            pltpu.VMEM((2, B, Sq, D_MODEL), jnp.bfloat16),
            pltpu.VMEM((2, B, Sq, D_MODEL), jnp.bfloat16),
            pltpu.SemaphoreType.DMA((2,)),
            pltpu.SemaphoreType.DMA((2,)),
        ],
        compiler_params=pltpu.CompilerParams(collective_id=0),
    )(x, Wq, k_loc, v_loc, Wo)


# device time: 8191 ns/iter; 2.9042x vs baseline; 2.2221x over previous
import jax
import jax.numpy as jnp
from jax import lax
from jax.experimental import pallas as pl
from jax.experimental.pallas import tpu as pltpu

B, Sq, Skv, Hq, Dh = 2, 128, 128, 16, 64
H_LOC = 4
D_MODEL = 512
HALF = D_MODEL // 2
BLK = 64


def kernel(x, Wq, K_ext, V_ext, Wo):
    def body(x_ref, wq_ref, k_hbm, v_hbm, wo_ref, out_ref,
             k_vm, v_vm, cp_sems, send_ref, recv_ref, send_sems, recv_sems):
        my_i = lax.axis_index("i")
        p1 = my_i ^ 1
        p2 = 3 - my_i

        barrier = pltpu.get_barrier_semaphore()
        for p in (p1, p2):
            pl.semaphore_signal(
                barrier, inc=1,
                device_id=(p,), device_id_type=pl.DeviceIdType.MESH,
            )

        kv_copies = []
        for b in range(B):
            for h in range(H_LOC):
                g = my_i * H_LOC + h
                ck = pltpu.make_async_copy(
                    k_hbm.at[b, :, g, :], k_vm.at[b, h], cp_sems.at[0])
                cv = pltpu.make_async_copy(
                    v_hbm.at[b, :, g, :], v_vm.at[b, h], cp_sems.at[1])
                ck.start()
                cv.start()
                kv_copies += [ck, cv]

        wqs = (wq_ref[...] * 0.125).astype(jnp.bfloat16)
        wo = wo_ref[...].astype(jnp.bfloat16)
        qs = [
            lax.dot(x_ref[b].astype(jnp.bfloat16), wqs,
                    preferred_element_type=jnp.float32).astype(jnp.bfloat16)
            for b in range(B)
        ]

        for c in kv_copies:
            c.wait()

        def exchange(step, b, half, p):
            lo = half * HALF
            r = pltpu.make_async_remote_copy(
                src_ref=send_ref.at[step, b, :, lo:lo + HALF],
                dst_ref=recv_ref.at[step, b, :, lo:lo + HALF],
                send_sem=send_sems.at[step, b, half],
                recv_sem=recv_sems.at[step, b, half],
                device_id=(p,),
                device_id_type=pl.DeviceIdType.MESH,
            )
            r.start()
            return r

        rdmas = []
        for b in range(B):
            q = qs[b]
            partial = None
            for h in range(H_LOC):
                qh = q[:, h * Dh:(h + 1) * Dh]
                kh = k_vm[b, h].astype(jnp.bfloat16)
                vh = v_vm[b, h].astype(jnp.bfloat16)
                s0 = lax.dot_general(
                    qh[0:BLK], kh[0:BLK], (((1,), (1,)), ((), ())),
                    preferred_element_type=jnp.float32)
                s1 = lax.dot_general(
                    qh[BLK:], kh, (((1,), (1,)), ((), ())),
                    preferred_element_type=jnp.float32)
                w0 = jnp.exp(s0)
                w1 = jnp.exp(s1)
                r0 = 1.0 / jnp.sum(w0, axis=-1, keepdims=True)
                r1 = 1.0 / jnp.sum(w1, axis=-1, keepdims=True)
                c0 = lax.dot(w0.astype(jnp.bfloat16), vh[0:BLK],
                             preferred_element_type=jnp.float32) * r0
                c1 = lax.dot(w1.astype(jnp.bfloat16), vh,
                             preferred_element_type=jnp.float32) * r1
                ctx = jnp.concatenate([c0, c1], axis=0).astype(jnp.bfloat16)
                d = lax.dot(ctx, wo[h * Dh:(h + 1) * Dh, :],
                            preferred_element_type=jnp.float32)
                partial = d if partial is None else partial + d
            out_ref[b] = partial
            send_ref[0, b] = partial.astype(jnp.bfloat16)
            if b == 0:
                pl.semaphore_wait(barrier, 2)
            rdmas.append(exchange(0, b, 0, p1))
            rdmas.append(exchange(0, b, 1, p2))

        for b in range(B):
            rdmas[2 * b].wait_recv()
            rdmas[2 * b + 1].wait_recv()
            acc = out_ref[b] + recv_ref[0, b].astype(jnp.float32)
            out_ref[b] = acc
            send_ref[1, b] = acc.astype(jnp.bfloat16)
            rdmas.append(exchange(1, b, 0, p2))
            rdmas.append(exchange(1, b, 1, p1))

        for b in range(B):
            rdmas[4 + 2 * b].wait_recv()
            rdmas[4 + 2 * b + 1].wait_recv()
            out_ref[b] = out_ref[b] + recv_ref[1, b].astype(jnp.float32)

        for r in rdmas:
            r.wait_send()

    return pl.pallas_call(
        body,
        out_shape=jax.ShapeDtypeStruct((B, Sq, D_MODEL), jnp.float32),
        in_specs=[
            pl.BlockSpec(memory_space=pltpu.VMEM),
            pl.BlockSpec(memory_space=pltpu.VMEM),
            pl.BlockSpec(memory_space=pltpu.MemorySpace.HBM),
            pl.BlockSpec(memory_space=pltpu.MemorySpace.HBM),
            pl.BlockSpec(memory_space=pltpu.VMEM),
        ],
        out_specs=pl.BlockSpec(memory_space=pltpu.VMEM),
        scratch_shapes=[
            pltpu.VMEM((B, H_LOC, Sq, Dh), jnp.float32),
            pltpu.VMEM((B, H_LOC, Sq, Dh), jnp.float32),
            pltpu.SemaphoreType.DMA((2,)),
            pltpu.VMEM((2, B, Sq, D_MODEL), jnp.bfloat16),
            pltpu.VMEM((2, B, Sq, D_MODEL), jnp.bfloat16),
            pltpu.SemaphoreType.DMA((2, B, 2)),
            pltpu.SemaphoreType.DMA((2, B, 2)),
        ],
        compiler_params=pltpu.CompilerParams(collective_id=0),
    )(x, Wq, K_ext, V_ext, Wo)


# device time: 6150 ns/iter; 3.8680x vs baseline; 1.3319x over previous
import jax
import jax.numpy as jnp
from jax import lax
from jax.experimental import pallas as pl
from jax.experimental.pallas import tpu as pltpu

B, Sq, Skv, Hq, Dh = 2, 128, 128, 16, 64
H_LOC = 4
D_MODEL = 512
HALF = D_MODEL // 2
BLK = 64


def kernel(x, Wq, K_ext, V_ext, Wo):
    my_pos = lax.axis_index("i")
    k_loc = jnp.transpose(
        lax.dynamic_slice_in_dim(K_ext, my_pos * H_LOC, H_LOC, axis=2),
        (0, 2, 1, 3))
    v_loc = jnp.transpose(
        lax.dynamic_slice_in_dim(V_ext, my_pos * H_LOC, H_LOC, axis=2),
        (0, 2, 1, 3))

    def body(x_ref, wq_ref, k_vm, v_vm, wo_ref, out_ref,
             send_ref, recv_ref, send_sems, recv_sems):
        my_i = lax.axis_index("i")
        p1 = my_i ^ 1
        p2 = 3 - my_i

        barrier = pltpu.get_barrier_semaphore()
        for p in (p1, p2):
            pl.semaphore_signal(
                barrier, inc=1,
                device_id=(p,), device_id_type=pl.DeviceIdType.MESH,
            )

        wqs = (wq_ref[...] * 0.125).astype(jnp.bfloat16)
        wo = wo_ref[...].astype(jnp.bfloat16)
        qs = [
            lax.dot(x_ref[b].astype(jnp.bfloat16), wqs,
                    preferred_element_type=jnp.float32).astype(jnp.bfloat16)
            for b in range(B)
        ]

        def exchange(step, b, half, p):
            lo = half * HALF
            r = pltpu.make_async_remote_copy(
                src_ref=send_ref.at[step, b, :, lo:lo + HALF],
                dst_ref=recv_ref.at[step, b, :, lo:lo + HALF],
                send_sem=send_sems.at[step, b, half],
                recv_sem=recv_sems.at[step, b, half],
                device_id=(p,),
                device_id_type=pl.DeviceIdType.MESH,
            )
            r.start()
            return r

        rdmas = []
        for b in range(B):
            q = qs[b]
            partial = None
            for h in range(H_LOC):
                qh = q[:, h * Dh:(h + 1) * Dh]
                kh = k_vm[b, h].astype(jnp.bfloat16)
                vh = v_vm[b, h].astype(jnp.bfloat16)
                s0 = lax.dot_general(
                    qh[0:BLK], kh[0:BLK], (((1,), (1,)), ((), ())),
                    preferred_element_type=jnp.float32)
                s1 = lax.dot_general(
                    qh[BLK:], kh, (((1,), (1,)), ((), ())),
                    preferred_element_type=jnp.float32)
                w0 = jnp.exp(s0)
                w1 = jnp.exp(s1)
                r0 = 1.0 / jnp.sum(w0, axis=-1, keepdims=True)
                r1 = 1.0 / jnp.sum(w1, axis=-1, keepdims=True)
                c0 = lax.dot(w0.astype(jnp.bfloat16), vh[0:BLK],
                             preferred_element_type=jnp.float32) * r0
                c1 = lax.dot(w1.astype(jnp.bfloat16), vh,
                             preferred_element_type=jnp.float32) * r1
                ctx = jnp.concatenate([c0, c1], axis=0).astype(jnp.bfloat16)
                d = lax.dot(ctx, wo[h * Dh:(h + 1) * Dh, :],
                            preferred_element_type=jnp.float32)
                partial = d if partial is None else partial + d
            out_ref[b] = partial
            send_ref[0, b] = partial.astype(jnp.bfloat16)
            if b == 0:
                pl.semaphore_wait(barrier, 2)
            rdmas.append(exchange(0, b, 0, p1))
            rdmas.append(exchange(0, b, 1, p2))

        for b in range(B):
            rdmas[2 * b].wait_recv()
            rdmas[2 * b + 1].wait_recv()
            acc = out_ref[b] + recv_ref[0, b].astype(jnp.float32)
            out_ref[b] = acc
            send_ref[1, b] = acc.astype(jnp.bfloat16)
            rdmas.append(exchange(1, b, 0, p2))
            rdmas.append(exchange(1, b, 1, p1))

        for b in range(B):
            rdmas[4 + 2 * b].wait_recv()
            rdmas[4 + 2 * b + 1].wait_recv()
            out_ref[b] = out_ref[b] + recv_ref[1, b].astype(jnp.float32)

        for r in rdmas:
            r.wait_send()

    return pl.pallas_call(
        body,
        out_shape=jax.ShapeDtypeStruct((B, Sq, D_MODEL), jnp.float32),
        in_specs=[
            pl.BlockSpec(memory_space=pltpu.VMEM),
            pl.BlockSpec(memory_space=pltpu.VMEM),
            pl.BlockSpec(memory_space=pltpu.VMEM),
            pl.BlockSpec(memory_space=pltpu.VMEM),
            pl.BlockSpec(memory_space=pltpu.VMEM),
        ],
        out_specs=pl.BlockSpec(memory_space=pltpu.VMEM),
        scratch_shapes=[
            pltpu.VMEM((2, B, Sq, D_MODEL), jnp.bfloat16),
            pltpu.VMEM((2, B, Sq, D_MODEL), jnp.bfloat16),
            pltpu.SemaphoreType.DMA((2, B, 2)),
            pltpu.SemaphoreType.DMA((2, B, 2)),
        ],
        compiler_params=pltpu.CompilerParams(collective_id=0),
    )(x, Wq, k_loc, v_loc, Wo)
